# baseline (speedup 1.0000x reference)
import functools

import jax
import jax.numpy as jnp
from jax.experimental import pallas as pl

_K1 = 42
_K2 = _K1 ^ 0x1BD11BDA
_ROT_A = (13, 15, 26, 6)
_ROT_B = (17, 29, 16, 24)
_INJ = (
    (_K1, (_K2 + 1) & 0xFFFFFFFF),
    (_K2, 2),
    (0, (_K1 + 3) & 0xFFFFFFFF),
    (_K1, (_K2 + 4) & 0xFFFFFFFF),
    (_K2, 5),
)

_TINY = float(jnp.finfo(jnp.float32).tiny)


def _gumbel(idx):
    def rotl(v, r):
        return (v << jnp.uint32(r)) | (v >> jnp.uint32(32 - r))

    x1 = idx + jnp.uint32(_K1)
    x0 = x1
    x1 = rotl(x1, 13) ^ x0
    for r in _ROT_A[1:]:
        x0 = x0 + x1
        x1 = rotl(x1, r)
        x1 = x0 ^ x1
    for g in range(5):
        if g > 0:
            rots = _ROT_A if g % 2 == 0 else _ROT_B
            for r in rots:
                x0 = x0 + x1
                x1 = rotl(x1, r)
                x1 = x0 ^ x1
        inj0, inj1 = _INJ[g]
        if inj0:
            x0 = x0 + jnp.uint32(inj0)
        x1 = x1 + jnp.uint32(inj1)
    bits = x0 ^ x1

    float_bits = (bits >> jnp.uint32(9)) | jnp.uint32(0x3F800000)
    floats = jax.lax.bitcast_convert_type(float_bits, jnp.float32) - jnp.float32(1.0)
    u = jnp.maximum(floats, jnp.float32(_TINY))
    return -jnp.log(-jnp.log(u))


def _pool_kernel(x_ref, s_ref, o_ref, *, rows_per_step, out_w, out_h):
    bk = x_ref.shape[0]
    base_oh = pl.program_id(1) * rows_per_step

    bc = pl.program_id(0) * bk + jax.lax.broadcasted_iota(
        jnp.int32, (bk, out_w), 0
    )
    ow4 = jax.lax.broadcasted_iota(jnp.int32, (bk, out_w), 1) << 2
    bc_term = (bc * (out_h * out_w * 4) + ow4).astype(jnp.uint32)

    sel = s_ref[...]

    def compact(vec):
        res = jax.lax.dot_general(
            vec,
            sel,
            (((1,), (0,)), ((), ())),
            precision=jax.lax.Precision.HIGHEST,
            preferred_element_type=jnp.float32,
        )
        return res[:, :out_w], res[:, 128 : 128 + out_w]

    for j in range(rows_per_step):
        r0 = x_ref[:, 2 * j, :]
        r1 = x_ref[:, 2 * j + 1, :]
        row_base = bc_term + jnp.uint32((base_oh + j) * out_w * 4)

        p0, p1 = compact(r0)
        p2, p3 = compact(r1)

        best_v = p0 + _gumbel(row_base)
        best_p = p0
        for j3, pj in ((1, p1), (2, p2), (3, p3)):
            aj = pj + _gumbel(row_base + jnp.uint32(j3))
            upd = aj > best_v
            best_v = jnp.where(upd, aj, best_v)
            best_p = jnp.where(upd, pj, best_p)
        o_ref[:, j, :] = best_p


def kernel(x):
    B, C, H, W = x.shape
    out_h, out_w = H // 2, W // 2
    BC = B * C
    xr = x.reshape(BC, H, W)

    bk = 256 if BC % 256 == 0 else (8 if BC % 8 == 0 else 1)
    rows_per_step = 8 if out_h % 8 == 0 else (4 if out_h % 4 == 0 else 1)
    grid = (BC // bk, out_h // rows_per_step)

    wi = jax.lax.broadcasted_iota(jnp.int32, (W, 256), 0)
    ki = jax.lax.broadcasted_iota(jnp.int32, (W, 256), 1)
    sel = ((wi == 2 * ki) | (wi == 2 * ki - 255)).astype(jnp.float32)

    body = functools.partial(
        _pool_kernel, rows_per_step=rows_per_step, out_w=out_w, out_h=out_h
    )
    out = pl.pallas_call(
        body,
        grid=grid,
        in_specs=[
            pl.BlockSpec((bk, 2 * rows_per_step, W), lambda i, j: (i, j, 0)),
            pl.BlockSpec((W, 256), lambda i, j: (0, 0)),
        ],
        out_specs=pl.BlockSpec(
            (bk, rows_per_step, out_w), lambda i, j: (i, j, 0)
        ),
        out_shape=jax.ShapeDtypeStruct((BC, out_h, out_w), jnp.float32),
    )(xr, sel)
    return out.reshape(B, C, out_h, out_w)


# speedup vs baseline: 1.0142x; 1.0142x over previous
import functools

import jax
import jax.numpy as jnp
from jax.experimental import pallas as pl

_K1 = 42
_K2 = _K1 ^ 0x1BD11BDA
_ROT_A = (13, 15, 26, 6)
_ROT_B = (17, 29, 16, 24)
_INJ = (
    (_K1, (_K2 + 1) & 0xFFFFFFFF),
    (_K2, 2),
    (0, (_K1 + 3) & 0xFFFFFFFF),
    (_K1, (_K2 + 4) & 0xFFFFFFFF),
    (_K2, 5),
)

_TINY = float(jnp.finfo(jnp.float32).tiny)


def _gumbel(idx):
    def rotl(v, r):
        return (v << jnp.uint32(r)) | (v >> jnp.uint32(32 - r))

    x1 = idx + jnp.uint32(_K1)
    x0 = x1
    x1 = rotl(x1, 13) ^ x0
    for r in _ROT_A[1:]:
        x0 = x0 + x1
        x1 = rotl(x1, r)
        x1 = x0 ^ x1
    for g in range(5):
        if g > 0:
            rots = _ROT_A if g % 2 == 0 else _ROT_B
            for r in rots:
                x0 = x0 + x1
                x1 = rotl(x1, r)
                x1 = x0 ^ x1
        inj0, inj1 = _INJ[g]
        if inj0:
            x0 = x0 + jnp.uint32(inj0)
        x1 = x1 + jnp.uint32(inj1)
    bits = x0 ^ x1

    float_bits = (bits >> jnp.uint32(9)) | jnp.uint32(0x3F800000)
    floats = jax.lax.bitcast_convert_type(float_bits, jnp.float32) - jnp.float32(1.0)
    u = jnp.maximum(floats, jnp.float32(_TINY))
    return -jnp.log(-jnp.log(u))


def _pool_kernel(x_ref, s_ref, o_ref, *, rows_per_step, out_w, out_h):
    bk = x_ref.shape[0]
    base_oh = pl.program_id(1) * rows_per_step

    bc = pl.program_id(0) * bk + jax.lax.broadcasted_iota(
        jnp.int32, (bk, out_w), 0
    )
    ow4 = jax.lax.broadcasted_iota(jnp.int32, (bk, out_w), 1) << 2
    bc_term = (bc * (out_h * out_w * 4) + ow4).astype(jnp.uint32)

    sel = s_ref[...]

    def compact(vec):
        res = jax.lax.dot_general(
            vec,
            sel,
            (((1,), (0,)), ((), ())),
            precision=jax.lax.Precision.HIGHEST,
            preferred_element_type=jnp.float32,
        )
        return res[:, :out_w], res[:, 128 : 128 + out_w]

    for j in range(rows_per_step):
        r0 = x_ref[:, 2 * j, :]
        r1 = x_ref[:, 2 * j + 1, :]
        row_base = bc_term + jnp.uint32((base_oh + j) * out_w * 4)

        p0, p1 = compact(r0)
        p2, p3 = compact(r1)

        best_v = p0 + _gumbel(row_base)
        best_p = p0
        for j3, pj in ((1, p1), (2, p2), (3, p3)):
            aj = pj + _gumbel(row_base + jnp.uint32(j3))
            upd = aj > best_v
            best_v = jnp.where(upd, aj, best_v)
            best_p = jnp.where(upd, pj, best_p)
        o_ref[:, j, :] = best_p


def kernel(x):
    B, C, H, W = x.shape
    out_h, out_w = H // 2, W // 2
    BC = B * C
    xr = x.reshape(BC, H, W)

    bk = 128 if BC % 128 == 0 else (8 if BC % 8 == 0 else 1)
    rows_per_step = 16 if out_h % 16 == 0 else (8 if out_h % 8 == 0 else 1)
    grid = (BC // bk, out_h // rows_per_step)

    wi = jax.lax.broadcasted_iota(jnp.int32, (W, 256), 0)
    ki = jax.lax.broadcasted_iota(jnp.int32, (W, 256), 1)
    sel = ((wi == 2 * ki) | (wi == 2 * ki - 255)).astype(jnp.float32)

    body = functools.partial(
        _pool_kernel, rows_per_step=rows_per_step, out_w=out_w, out_h=out_h
    )
    out = pl.pallas_call(
        body,
        grid=grid,
        in_specs=[
            pl.BlockSpec((bk, 2 * rows_per_step, W), lambda i, j: (i, j, 0)),
            pl.BlockSpec((W, 256), lambda i, j: (0, 0)),
        ],
        out_specs=pl.BlockSpec(
            (bk, rows_per_step, out_w), lambda i, j: (i, j, 0)
        ),
        out_shape=jax.ShapeDtypeStruct((BC, out_h, out_w), jnp.float32),
    )(xr, sel)
    return out.reshape(B, C, out_h, out_w)
